# baseline (speedup 1.0000x reference)
import functools

import jax
import jax.numpy as jnp
from jax import lax
from jax.experimental import pallas as pl
from jax.experimental.pallas import tpu as pltpu
from jax.experimental.pallas import tpu_sc as plsc

L = 16

_GATHER_DNUMS = lax.GatherDimensionNumbers(
    offset_dims=(), collapsed_slice_dims=(0,), start_index_map=(0,))


def _butterfly_sum(r, lanes):
    for k in (1, 2, 4, 8):
        perm = jnp.bitwise_xor(lanes, k).reshape(L, 1)
        r = r + lax.gather(r, perm, _GATHER_DNUMS, (1,),
                           mode=lax.GatherScatterMode.PROMISE_IN_BOUNDS)
    return r


def _fold_weights(W1, W2, W3, b1, b2, b3, fm_bias):
    def body(w1_ref, w2_ref, w3_ref, b1_ref, b2_ref, b3_ref, fmb_ref,
             weff_ref, btot_ref):
        w23 = jnp.dot(w2_ref[...], w3_ref[...],
                      preferred_element_type=jnp.float32)
        weff_ref[...] = jnp.dot(w1_ref[...], w23,
                                preferred_element_type=jnp.float32)
        btot = (jnp.dot(b1_ref[...], w23, preferred_element_type=jnp.float32)
                + jnp.dot(b2_ref[...], w3_ref[...],
                          preferred_element_type=jnp.float32))
        btot_ref[...] = btot + b3_ref[...] + fmb_ref[...]

    fe = W1.shape[0]
    weff, btot = pl.pallas_call(
        body,
        out_shape=(
            jax.ShapeDtypeStruct((fe, 1), jnp.float32),
            jax.ShapeDtypeStruct((1, 1), jnp.float32),
        ),
    )(W1, W2, W3, b1.reshape(1, -1), b2.reshape(1, -1), b3.reshape(1, 1),
      jnp.reshape(fm_bias, (1, 1)).astype(jnp.float32))
    return weff, btot


def _make_sc_kernel(B, F, E, NW):
    SPW = B // NW
    NG = SPW // L
    mesh = plsc.VectorSubcoreMesh(core_axis_name="c", subcore_axis_name="s")

    @functools.partial(
        pl.kernel,
        out_type=jax.ShapeDtypeStruct((B,), jnp.float32),
        mesh=mesh,
        scratch_types=[
            pltpu.VMEM((F, SPW), jnp.int32),
            pltpu.VMEM((F * SPW,), jnp.float32),
            pltpu.VMEM((F, SPW, E), jnp.float32),
            pltpu.VMEM((F, SPW), jnp.float32),
            pltpu.VMEM((F, E), jnp.float32),
            pltpu.VMEM((L,), jnp.float32),
            pltpu.VMEM((SPW,), jnp.float32),
            pltpu.SemaphoreType.DMA,
            pltpu.SemaphoreType.DMA,
        ],
        compiler_params=pltpu.CompilerParams(use_tc_tiling_on_sc=False),
    )
    def sc_kernel(idx_hbm, vals_hbm, emb_hbm, fmw_hbm, weff_hbm, btot_hbm,
                  out_hbm, idx_v, vals_v, rows_v, fw_v, weff_v, btot_v, out_v,
                  sem_rows, sem_fw):
        wid = lax.axis_index("s") * 2 + lax.axis_index("c")
        base = pl.multiple_of(wid * SPW, SPW)

        pltpu.sync_copy(idx_hbm.at[wid], idx_v)

        handles = []
        for f in range(F):
            handles.append(
                pltpu.async_copy(emb_hbm.at[idx_v.at[f]], rows_v.at[f],
                                 sem_rows))
            handles.append(
                pltpu.async_copy(fmw_hbm.at[idx_v.at[f]], fw_v.at[f], sem_fw))
        pltpu.sync_copy(vals_hbm.at[wid], vals_v)
        pltpu.sync_copy(weff_hbm, weff_v)
        pltpu.sync_copy(btot_hbm, btot_v)
        for h in handles:
            h.wait()

        lanes = lax.iota(jnp.int32, L)
        zero = jnp.zeros((L,), jnp.float32)
        btot = btot_v[...]

        def group_body(g, _):
            s0 = pl.multiple_of(g * L, L)

            def pair_body(p, outz):
                la = 2 * p
                lb = la + 1
                sa = s0 + la
                sb = sa + 1
                lidx_a = jnp.full((L, 1), la, jnp.int32)
                lidx_b = jnp.full((L, 1), lb, jnp.int32)
                aa0 = aa1 = qa0 = qa1 = da0 = da1 = zero
                ab0 = ab1 = qb0 = qb1 = db0 = db1 = zero
                for f in range(F):
                    w0 = weff_v[f, pl.ds(0, L)]
                    w1 = weff_v[f, pl.ds(L, L)]
                    vchunk = vals_v[pl.ds(f * SPW + s0, L)]
                    va = lax.gather(
                        vchunk, lidx_a, _GATHER_DNUMS, (1,),
                        mode=lax.GatherScatterMode.PROMISE_IN_BOUNDS)
                    vb = lax.gather(
                        vchunk, lidx_b, _GATHER_DNUMS, (1,),
                        mode=lax.GatherScatterMode.PROMISE_IN_BOUNDS)
                    ea0 = rows_v[f, sa, pl.ds(0, L)]
                    ea1 = rows_v[f, sa, pl.ds(L, L)]
                    eb0 = rows_v[f, sb, pl.ds(0, L)]
                    eb1 = rows_v[f, sb, pl.ds(L, L)]
                    sa0 = ea0 * va
                    sa1 = ea1 * va
                    sb0 = eb0 * vb
                    sb1 = eb1 * vb
                    aa0 = aa0 + sa0
                    aa1 = aa1 + sa1
                    ab0 = ab0 + sb0
                    ab1 = ab1 + sb1
                    qa0 = qa0 + sa0 * sa0
                    qa1 = qa1 + sa1 * sa1
                    qb0 = qb0 + sb0 * sb0
                    qb1 = qb1 + sb1 * sb1
                    da0 = da0 + sa0 * w0
                    da1 = da1 + sa1 * w1
                    db0 = db0 + sb0 * w0
                    db1 = db1 + sb1 * w1
                ra = da0 + da1 + 0.5 * (aa0 * aa0 + aa1 * aa1 - qa0 - qa1)
                rb = db0 + db1 + 0.5 * (ab0 * ab0 + ab1 * ab1 - qb0 - qb1)
                ra = _butterfly_sum(ra, lanes)
                rb = _butterfly_sum(rb, lanes)
                outz = jnp.where(lanes == la, ra, outz)
                return jnp.where(lanes == lb, rb, outz)

            outz = lax.fori_loop(0, L // 2, pair_body, zero)

            fm1 = zero
            for f in range(F):
                fm1 = fm1 + fw_v[f, pl.ds(s0, L)] * vals_v[pl.ds(f * SPW + s0, L)]

            zv = outz + fm1 + btot
            out_v[pl.ds(s0, L)] = 1.0 / (1.0 + jnp.exp(-zv))
            return 0

        lax.fori_loop(0, NG, group_body, 0)
        pltpu.sync_copy(out_v, out_hbm.at[pl.ds(base, SPW)])

    return sc_kernel


def kernel(idxs, vals, shared_emb_table, fm_w_table, fm_bias,
           W1, b1, W2, b2, W3, b3):
    B, F = idxs.shape
    V, E = shared_emb_table.shape
    NW = 32
    SPW = B // NW

    weff, btot = _fold_weights(W1, W2, W3, b1, b2, b3, fm_bias)

    idx_w = idxs.reshape(NW, SPW, F).transpose(0, 2, 1)
    vals_w = vals.reshape(NW, SPW, F).transpose(0, 2, 1).reshape(NW, F * SPW)

    sc = _make_sc_kernel(B, F, E, NW)
    out_flat = sc(idx_w, vals_w, shared_emb_table, fm_w_table.reshape(-1),
                  weff.reshape(F, E), jnp.broadcast_to(btot.reshape(1), (L,)))
    return out_flat.reshape(B, 1)
